# baseline (speedup 1.0000x reference)
import functools

import jax
import jax.numpy as jnp
from jax import lax
from jax.experimental import pallas as pl
from jax.experimental.pallas import tpu as pltpu, tpu_sc as plsc

V = 1000000
F = 26
D = 13
E = 16
H1 = 64
H2 = 32
B = 4096
FE = F * E
BF = B * F

_NC, _NS = 2, 16
_NW = _NC * _NS
_BPW = BF // _NW
_CH = _BPW // 128


@functools.cache
def _sc_gather_fn():
    mesh = plsc.VectorSubcoreMesh(core_axis_name="c", subcore_axis_name="s")

    @functools.partial(
        pl.kernel,
        out_type=[
            jax.ShapeDtypeStruct((B, 512), jnp.float32),
            jax.ShapeDtypeStruct((BF // 128, 128), jnp.float32),
        ],
        mesh=mesh,
        scratch_types=[
            pltpu.VMEM((_CH, 128), jnp.int32),
            pltpu.VMEM((_BPW, E), jnp.float32),
            pltpu.VMEM((_BPW // F, 512), jnp.float32),
            pltpu.VMEM((_CH, 128), jnp.float32),
            pltpu.SemaphoreType.DMA,
            pltpu.SemaphoreType.DMA,
        ],
        compiler_params=pltpu.CompilerParams(use_tc_tiling_on_sc=False),
    )
    def _sc_gather(emb_hbm, lin_hbm, idx_hbm, e_out, lin_out,
                   idx_v, rows_v, rows2_v, lin_v, sem_e, sem_l):
        wid = lax.axis_index("s") * _NC + lax.axis_index("c")
        pltpu.sync_copy(idx_hbm.at[pl.ds(wid * _CH, _CH)], idx_v)

        def fire(j, _):
            pltpu.async_copy(
                emb_hbm.at[idx_v.at[j]], rows_v.at[pl.ds(j * 128, 128)], sem_e)
            pltpu.async_copy(lin_hbm.at[idx_v.at[j]], lin_v.at[j], sem_l)
            return 0

        lax.fori_loop(0, _CH, fire, 0)

        def drain(j, _):
            pltpu.make_async_copy(
                emb_hbm.at[idx_v.at[j]], rows_v.at[pl.ds(j * 128, 128)],
                sem_e).wait()
            pltpu.make_async_copy(
                lin_hbm.at[idx_v.at[j]], lin_v.at[j], sem_l).wait()
            return 0

        lax.fori_loop(0, _CH, drain, 0)

        zeros16 = jnp.zeros((E,), jnp.float32)

        def shuf(b, _):
            for c in range(F):
                rows2_v[b, pl.ds(c * E, E)] = rows_v[b * F + c, :]
            for c in range(F, 512 // E):
                rows2_v[b, pl.ds(c * E, E)] = zeros16
            return 0

        lax.fori_loop(0, _BPW // F, shuf, 0)
        pltpu.sync_copy(
            rows2_v, e_out.at[pl.ds(wid * (_BPW // F), _BPW // F)])
        pltpu.sync_copy(lin_v, lin_out.at[pl.ds(wid * _CH, _CH)])

    return _sc_gather


def _tc_body(e_ref, ling_ref, dense_ref, wdt_ref, bd_ref, w1et_ref, w1dt_ref,
             b1_ref, g1_ref, be1_ref, w2t_ref, b2_ref, g2_ref, be2_ref,
             wot_ref, bo_ref, out_ref):
    e = e_ref[...]
    dense = dense_ref[...]

    lin = (jnp.sum(ling_ref[...], axis=1, keepdims=True)
           + dense @ wdt_ref[...] + bd_ref[...])

    i1 = lax.broadcasted_iota(jnp.int32, (512, E), 0)
    i2 = lax.broadcasted_iota(jnp.int32, (512, E), 1)
    smat = jnp.where(i1 % E == i2, 1.0, 0.0)
    s = jax.lax.dot(e, smat, preferred_element_type=jnp.float32)
    sq = jax.lax.dot(e * e, smat, preferred_element_type=jnp.float32)
    fm = 0.5 * jnp.sum(s * s - sq, axis=1, keepdims=True)

    def bn(h, g, b, eps=1e-5):
        m = jnp.mean(h, axis=0, keepdims=True)
        v = jnp.mean(h * h, axis=0, keepdims=True) - m * m
        return (h - m) * jax.lax.rsqrt(v + eps) * g + b

    h1 = e @ w1et_ref[...] + dense @ w1dt_ref[...] + b1_ref[...]
    h1 = jnp.maximum(bn(h1, g1_ref[...], be1_ref[...]), 0.0)
    h2 = h1 @ w2t_ref[...] + b2_ref[...]
    h2 = jnp.maximum(bn(h2, g2_ref[...], be2_ref[...]), 0.0)
    logit = h2 @ wot_ref[...] + bo_ref[...] + lin + fm
    out_ref[...] = jax.nn.sigmoid(logit)


def kernel(sparse_inputs, dense_inputs, emb, lin_emb, Wd, bd, W1, b1, g1,
           be1, W2, b2, g2, be2, Wo, bo):
    idx2d = sparse_inputs.reshape(-1).astype(jnp.int32).reshape(BF // 128, 128)
    e_out, lin_out = _sc_gather_fn()(emb, lin_emb.reshape(V), idx2d)

    out = pl.pallas_call(
        _tc_body,
        out_shape=jax.ShapeDtypeStruct((B, 1), jnp.float32),
    )(
        e_out,
        lin_out.reshape(B, F),
        dense_inputs,
        Wd.T,
        bd.reshape(1, 1),
        jnp.pad(W1[:, :FE].T, ((0, 512 - FE), (0, 0))),
        W1[:, FE:].T,
        b1.reshape(1, H1),
        g1.reshape(1, H1),
        be1.reshape(1, H1),
        W2.T,
        b2.reshape(1, H2),
        g2.reshape(1, H2),
        be2.reshape(1, H2),
        Wo.T,
        bo.reshape(1, 1),
    )
    return out


# speedup vs baseline: 1.3767x; 1.3767x over previous
import functools

import jax
import jax.numpy as jnp
from jax import lax
from jax.experimental import pallas as pl
from jax.experimental.pallas import tpu as pltpu, tpu_sc as plsc
from jax.experimental import layout as jex_layout

V = 1000000
F = 26
D = 13
E = 16
H1 = 64
H2 = 32
B = 4096
FE = F * E
BF = B * F

_NC, _NS = 2, 16
_NW = _NC * _NS
_BPW = BF // _NW
_CH = _BPW // 128


@functools.cache
def _sc_gather_fn():
    mesh = plsc.VectorSubcoreMesh(core_axis_name="c", subcore_axis_name="s")

    @functools.partial(
        pl.kernel,
        out_type=[
            jax.ShapeDtypeStruct((B, 512), jnp.float32),
            jax.ShapeDtypeStruct((BF // 128, 128), jnp.float32),
        ],
        mesh=mesh,
        scratch_types=[
            pltpu.VMEM((_CH, 128), jnp.int32),
            pltpu.VMEM((_BPW, E), jnp.float32),
            pltpu.VMEM((_BPW // F, 512), jnp.float32),
            pltpu.VMEM((_CH, 128), jnp.float32),
            pltpu.SemaphoreType.DMA,
            pltpu.SemaphoreType.DMA,
        ],
        compiler_params=pltpu.CompilerParams(use_tc_tiling_on_sc=False),
    )
    def _sc_gather(emb_hbm, lin_hbm, idx_hbm, e_out, lin_out,
                   idx_v, rows_v, rows2_v, lin_v, sem_e, sem_l):
        wid = lax.axis_index("s") * _NC + lax.axis_index("c")
        pltpu.sync_copy(idx_hbm.at[pl.ds(wid * _CH, _CH)], idx_v)

        def fire(j, _):
            pltpu.async_copy(
                emb_hbm.at[idx_v.at[j]], rows_v.at[pl.ds(j * 128, 128)], sem_e)
            pltpu.async_copy(lin_hbm.at[idx_v.at[j]], lin_v.at[j], sem_l)
            return 0

        lax.fori_loop(0, _CH, fire, 0)

        def drain(j, _):
            pltpu.make_async_copy(
                emb_hbm.at[idx_v.at[j]], rows_v.at[pl.ds(j * 128, 128)],
                sem_e).wait()
            pltpu.make_async_copy(
                lin_hbm.at[idx_v.at[j]], lin_v.at[j], sem_l).wait()
            return 0

        lax.fori_loop(0, _CH, drain, 0)

        zeros16 = jnp.zeros((E,), jnp.float32)

        def shuf(b, _):
            for c in range(F):
                rows2_v[b, pl.ds(c * E, E)] = rows_v[b * F + c, :]
            for c in range(F, 512 // E):
                rows2_v[b, pl.ds(c * E, E)] = zeros16
            return 0

        lax.fori_loop(0, _BPW // F, shuf, 0)
        pltpu.sync_copy(
            rows2_v, e_out.at[pl.ds(wid * (_BPW // F), _BPW // F)])
        pltpu.sync_copy(lin_v, lin_out.at[pl.ds(wid * _CH, _CH)])

    return _sc_gather


def _tc_body(e_ref, ling_ref, dense_ref, wdt_ref, bd_ref, w1et_ref, w1dt_ref,
             b1_ref, g1_ref, be1_ref, w2t_ref, b2_ref, g2_ref, be2_ref,
             wot_ref, bo_ref, out_ref):
    e = e_ref[...]
    dense = dense_ref[...]

    lin = (jnp.sum(ling_ref[...], axis=1, keepdims=True)
           + dense @ wdt_ref[...] + bd_ref[...])

    i1 = lax.broadcasted_iota(jnp.int32, (512, E), 0)
    i2 = lax.broadcasted_iota(jnp.int32, (512, E), 1)
    smat = jnp.where(i1 % E == i2, 1.0, 0.0)
    s = jax.lax.dot(e, smat, preferred_element_type=jnp.float32)
    sq = jax.lax.dot(e * e, smat, preferred_element_type=jnp.float32)
    fm = 0.5 * jnp.sum(s * s - sq, axis=1, keepdims=True)

    def bn(h, g, b, eps=1e-5):
        m = jnp.mean(h, axis=0, keepdims=True)
        v = jnp.mean(h * h, axis=0, keepdims=True) - m * m
        return (h - m) * jax.lax.rsqrt(v + eps) * g + b

    h1 = e @ w1et_ref[...] + dense @ w1dt_ref[...] + b1_ref[...]
    h1 = jnp.maximum(bn(h1, g1_ref[...], be1_ref[...]), 0.0)
    h2 = h1 @ w2t_ref[...] + b2_ref[...]
    h2 = jnp.maximum(bn(h2, g2_ref[...], be2_ref[...]), 0.0)
    logit = h2 @ wot_ref[...] + bo_ref[...] + lin + fm
    out_ref[...] = jax.nn.sigmoid(logit)


def kernel(sparse_inputs, dense_inputs, emb, lin_emb, Wd, bd, W1, b1, g1,
           be1, W2, b2, g2, be2, Wo, bo):
    idx2d = sparse_inputs.reshape(-1).astype(jnp.int32).reshape(BF // 128, 128)
    emb_rm = jex_layout.with_layout_constraint(
        emb,
        jex_layout.Layout(major_to_minor=(0, 1), tiling=()),
    )
    e_out, lin_out = _sc_gather_fn()(emb_rm, lin_emb.reshape(V), idx2d)

    out = pl.pallas_call(
        _tc_body,
        out_shape=jax.ShapeDtypeStruct((B, 1), jnp.float32),
    )(
        e_out,
        lin_out.reshape(B, F),
        dense_inputs,
        Wd.T,
        bd.reshape(1, 1),
        jnp.pad(W1[:, :FE].T, ((0, 512 - FE), (0, 0))),
        W1[:, FE:].T,
        b1.reshape(1, H1),
        g1.reshape(1, H1),
        be1.reshape(1, H1),
        W2.T,
        b2.reshape(1, H2),
        g2.reshape(1, H2),
        be2.reshape(1, H2),
        Wo.T,
        bo.reshape(1, 1),
    )
    return out
